# baseline (speedup 1.0000x reference)
import functools

import numpy as np

import jax
import jax.numpy as jnp
from jax.experimental import pallas as pl
from jax.experimental.pallas import tpu as pltpu

_WIN = 10
_BANDS = 6
_EPS = 1e-6
_HO = 124
_KP = 104
_NPIX = _HO * 128


def _plan():
    lin = jnp.linspace(-1.0, 1.0, _WIN)
    yy, xx = jnp.meshgrid(lin, lin, indexing='ij')
    rr = jnp.sqrt(xx * xx + yy * yy)
    rr = rr / jnp.maximum(rr.max(), 1e-6)
    edges = jnp.linspace(0.0, 1.0, _BANDS + 1)
    masks = ((rr[None] >= edges[:-1, None, None]) &
             (rr[None] < edges[1:, None, None]))
    masks_np = np.asarray(masks)
    band = np.full((_WIN, _WIN), -1, np.int64)
    counts = np.zeros(_BANDS, np.float64)
    for k in range(_BANDS):
        m = masks_np[k]
        band[m] = k
        counts[k] = max(m.sum(), 1.0)

    def twin(s):
        return ((_WIN - (s + _WIN // 2) % _WIN) % _WIN + _WIN // 2) % _WIN

    per_band = {k: [] for k in range(_BANDS)}
    for sy in range(_WIN):
        for sx in range(_WIN):
            u = sy * _WIN + sx
            t = twin(sy) * _WIN + twin(sx)
            if u > t:
                continue
            bu, bt = band[sy, sx], band[twin(sy), twin(sx)]
            if u == t:
                if bu >= 0:
                    per_band[bu].append((u, 1.0))
            elif bu == bt:
                if bu >= 0:
                    per_band[bu].append((u, 2.0))
            else:
                if bu >= 0:
                    per_band[bu].append((u, 1.0))
                if bt >= 0:
                    per_band[bt].append((u, 1.0))

    s = np.arange(_WIN)
    k = (s + _WIN // 2) % _WIN
    w = np.arange(_WIN)
    ang = 2.0 * np.pi * np.outer(k, w) / _WIN
    th = (ang[:, None, :, None] + ang[None, :, None, :]).reshape(100, 100)

    nrows = sum(len(v) for v in per_band.values())
    cs = np.zeros((2 * nrows, _WIN * _WIN), np.float64)
    segments = []
    i = 0
    for b in range(_BANDS):
        start = i
        for (u, scale) in per_band[b]:
            sc = scale / (10.0 * counts[b])
            cs[i] = sc * np.cos(th[u])
            cs[nrows + i] = sc * np.sin(th[u])
            i += 1
        segments.append((start, i))
    return cs.astype(np.float32), nrows, tuple(segments)


_CS, _NROWS, _SEGMENTS = _plan()


def _build_body(xp_ref, pt_ref):
    g = [[None, None], [None, None]]
    for py in range(2):
        for px in range(2):
            g[py][px] = (0.2989 * xp_ref[0, py, px, 0]
                         + 0.587 * xp_ref[0, py, px, 1]
                         + 0.114 * xp_ref[0, py, px, 2])

    for wy in range(_WIN):
        py, dy = wy % 2, wy // 2
        for wx in range(_WIN):
            px, dx = wx % 2, wx // 2
            pt_ref[0, wy * _WIN + wx, :, 0:_HO] = (
                g[py][px][dy:dy + _HO, dx:dx + _HO])
    pt_ref[0, :, :, _HO:128] = jnp.zeros((_KP, _HO, 128 - _HO), jnp.float32)
    pt_ref[0, 100:_KP, :, 0:_HO] = jnp.zeros((4, _HO, _HO), jnp.float32)


def _dft_body(pt_ref, cs3_ref, out_ref, *, nrows, segments):
    cs3 = cs3_ref[...]
    p = pt_ref[0]
    ph_f = pltpu.bitcast(
        pltpu.bitcast(p, jnp.uint32) & jnp.uint32(0xFFFF0000),
        jnp.float32)
    pl_f = p - ph_f
    p3 = jnp.concatenate([ph_f, pl_f, ph_f], axis=0)
    reim = jnp.dot(cs3, p3, preferred_element_type=jnp.float32)
    re = reim[0:nrows]
    im = reim[nrows:2 * nrows]
    amp = jnp.sqrt(re * re + im * im)
    stats = [jnp.sum(amp[a:b], axis=0) for (a, b) in segments]
    zz = jnp.zeros_like(stats[0])
    stat = jnp.stack(stats + [zz, zz], axis=0)
    out_ref[0] = jnp.log10(stat + _EPS)


@jax.jit
def kernel(x):
    b = x.shape[0]
    xp = x.reshape(b, 3, 128, 2, 128, 2).transpose(0, 3, 5, 1, 2, 4)
    nrows, segments = _NROWS, _SEGMENTS
    cs = jnp.zeros((2 * nrows, _KP), jnp.float32).at[:, :100].set(
        jnp.asarray(_CS))
    cs_hi = cs.astype(jnp.bfloat16).astype(jnp.float32)
    cs_lo = (cs - cs_hi).astype(jnp.bfloat16).astype(jnp.float32)
    cs3 = jnp.concatenate([cs_hi, cs_hi, cs_lo], axis=1)

    pt = pl.pallas_call(
        _build_body,
        grid=(b,),
        in_specs=[
            pl.BlockSpec((1, 2, 2, 3, 128, 128), lambda i: (i, 0, 0, 0, 0, 0)),
        ],
        out_specs=pl.BlockSpec((1, _KP, _HO, 128), lambda i: (i, 0, 0, 0)),
        out_shape=jax.ShapeDtypeStruct((b, _KP, _HO, 128), jnp.float32),
        compiler_params=pltpu.CompilerParams(
            dimension_semantics=("arbitrary",),
            vmem_limit_bytes=56 * 1024 * 1024,
        ),
    )(xp)

    pt2 = pt.reshape(b, _KP, _NPIX)

    nblk = 3968
    dft = functools.partial(_dft_body, nrows=nrows, segments=segments)
    out = pl.pallas_call(
        dft,
        grid=(b, _NPIX // nblk),
        in_specs=[
            pl.BlockSpec((1, _KP, nblk), lambda i, j: (i, 0, j)),
            pl.BlockSpec((2 * nrows, 312), lambda i, j: (0, 0)),
        ],
        out_specs=pl.BlockSpec((1, 8, nblk), lambda i, j: (i, 0, j)),
        out_shape=jax.ShapeDtypeStruct((b, 8, _NPIX), jnp.float32),
        compiler_params=pltpu.CompilerParams(
            dimension_semantics=("arbitrary", "arbitrary"),
            vmem_limit_bytes=56 * 1024 * 1024,
        ),
    )(pt2, cs3)

    return out.reshape(b, 8, _HO, 128)[:, :_BANDS, :, :_HO]


# speedup vs baseline: 1.3184x; 1.3184x over previous
import functools

import numpy as np

import jax
import jax.numpy as jnp
from jax.experimental import pallas as pl
from jax.experimental.pallas import tpu as pltpu

_WIN = 10
_BANDS = 6
_EPS = 1e-6
_HO = 124


def _plan():
    lin = jnp.linspace(-1.0, 1.0, _WIN)
    yy, xx = jnp.meshgrid(lin, lin, indexing='ij')
    rr = jnp.sqrt(xx * xx + yy * yy)
    rr = rr / jnp.maximum(rr.max(), 1e-6)
    edges = jnp.linspace(0.0, 1.0, _BANDS + 1)
    masks = ((rr[None] >= edges[:-1, None, None]) &
             (rr[None] < edges[1:, None, None]))
    masks_np = np.asarray(masks)
    band = np.full((_WIN, _WIN), -1, np.int64)
    counts = np.zeros(_BANDS, np.float64)
    for k in range(_BANDS):
        m = masks_np[k]
        band[m] = k
        counts[k] = max(m.sum(), 1.0)

    def twin(s):
        return ((_WIN - (s + _WIN // 2) % _WIN) % _WIN + _WIN // 2) % _WIN

    per_band = {k: [] for k in range(_BANDS)}
    for sy in range(_WIN):
        for sx in range(_WIN):
            u = sy * _WIN + sx
            t = twin(sy) * _WIN + twin(sx)
            if u > t:
                continue
            bu, bt = band[sy, sx], band[twin(sy), twin(sx)]
            if u == t:
                if bu >= 0:
                    per_band[bu].append((u, 1.0))
            elif bu == bt:
                if bu >= 0:
                    per_band[bu].append((u, 2.0))
            else:
                if bu >= 0:
                    per_band[bu].append((u, 1.0))
                if bt >= 0:
                    per_band[bt].append((u, 1.0))

    s = np.arange(_WIN)
    k = (s + _WIN // 2) % _WIN
    w = np.arange(_WIN)
    ang = 2.0 * np.pi * np.outer(k, w) / _WIN
    th = (ang[:, None, :, None] + ang[None, :, None, :]).reshape(100, 100)

    nrows = sum(len(v) for v in per_band.values())
    cs = np.zeros((2 * nrows, _WIN * _WIN), np.float64)
    segments = []
    i = 0
    for b in range(_BANDS):
        start = i
        for (u, scale) in per_band[b]:
            sc = scale / (10.0 * counts[b])
            cs[i] = sc * np.cos(th[u])
            cs[nrows + i] = sc * np.sin(th[u])
            i += 1
        segments.append((start, i))
    return cs.astype(np.float32), nrows, tuple(segments)


_CS, _NROWS, _SEGMENTS = _plan()


def _body(xp_ref, cs3_ref, out_ref, pt_ref, *, nrows, segments):
    g = [[None, None], [None, None]]
    for py in range(2):
        for px in range(2):
            g[py][px] = (0.2989 * xp_ref[0, py, px, 0]
                         + 0.587 * xp_ref[0, py, px, 1]
                         + 0.114 * xp_ref[0, py, px, 2])

    for wy in range(_WIN):
        py, dy = wy % 2, wy // 2
        for wx in range(_WIN):
            px, dx = wx % 2, wx // 2
            pt_ref[wy * _WIN + wx, :, 0:_HO] = (
                g[py][px][dy:dy + _HO, dx:dx + _HO])
    pt_ref[:, :, _HO:128] = jnp.zeros((104, _HO, 128 - _HO), jnp.float32)
    pt_ref[100:104, :, 0:_HO] = jnp.zeros((4, _HO, _HO), jnp.float32)

    cs3 = cs3_ref[...]
    for hc in range(0, _HO, 8):
        ch = min(8, _HO - hc)
        ptc = pt_ref[:, hc:hc + ch, :]
        ph_f = pltpu.bitcast(
            pltpu.bitcast(ptc, jnp.uint32) & jnp.uint32(0xFFFF0000),
            jnp.float32)
        pl_f = ptc - ph_f
        p3 = jnp.concatenate([ph_f, pl_f, ph_f], axis=0)
        reim = jnp.einsum('fo,ohw->fhw', cs3, p3,
                          preferred_element_type=jnp.float32)
        re = reim[0:nrows]
        im = reim[nrows:2 * nrows]
        amp = jnp.sqrt(re * re + im * im)
        stats = [jnp.sum(amp[a:b], axis=0) for (a, b) in segments]
        zz = jnp.zeros_like(stats[0])
        stat = jnp.stack(stats + [zz, zz], axis=0)
        out_ref[0, :, hc:hc + ch, :] = jnp.log10(stat + _EPS)


@jax.jit
def kernel(x):
    b = x.shape[0]
    xp = x.reshape(b, 3, 128, 2, 128, 2).transpose(0, 3, 5, 1, 2, 4)
    nrows, segments = _NROWS, _SEGMENTS
    cs = jnp.zeros((2 * nrows, 104), jnp.float32).at[:, :100].set(
        jnp.asarray(_CS))
    cs_hi = cs.astype(jnp.bfloat16).astype(jnp.float32)
    cs_lo = (cs - cs_hi).astype(jnp.bfloat16).astype(jnp.float32)
    cs3 = jnp.concatenate([cs_hi, cs_hi, cs_lo], axis=1)
    body = functools.partial(_body, nrows=nrows, segments=segments)
    out = pl.pallas_call(
        body,
        grid=(b,),
        in_specs=[
            pl.BlockSpec((1, 2, 2, 3, 128, 128), lambda i: (i, 0, 0, 0, 0, 0)),
            pl.BlockSpec((2 * nrows, 312), lambda i: (0, 0)),
        ],
        out_specs=pl.BlockSpec((1, 8, _HO, 128), lambda i: (i, 0, 0, 0)),
        out_shape=jax.ShapeDtypeStruct((b, 8, _HO, 128), jnp.float32),
        scratch_shapes=[pltpu.VMEM((104, _HO, 128), jnp.float32)],
        compiler_params=pltpu.CompilerParams(
            dimension_semantics=("arbitrary",),
            vmem_limit_bytes=56 * 1024 * 1024,
        ),
    )(xp, cs3)
    return out[:, :_BANDS, :, :_HO]
